# Initial kernel scaffold; baseline (speedup 1.0000x reference)
import jax
import jax.numpy as jnp
from jax.experimental import pallas as pl


def kernel(indices, hist, table_sparse, table_seq):
    raise NotImplementedError("write your pallas kernel here")


# speedup vs baseline: 4.1615x; 4.1615x over previous
import functools

import numpy as np
import jax
import jax.numpy as jnp
from jax import lax
from jax.experimental import pallas as pl
from jax.experimental.pallas import tpu as pltpu
from jax.experimental.pallas import tpu_sc as plsc

B, F, L, V, D = 4096, 26, 50, 100000, 64
NC, NS = 2, 16
NW = NC * NS
BPW = B // NW
SP_STRIPS = BPW * F // 128
SQ_STRIPS = BPW * L // 128
SP_PAD = 32
SQ_PAD = 56
NQ = D // 16


@functools.lru_cache(maxsize=1)
def _dest_arrays():
    i = np.arange(B * F, dtype=np.int32)
    sdst = ((i // F) * (F + 1) + (i % F)).astype(np.int32).reshape(NW, SP_STRIPS, 128)
    sdst = np.pad(sdst, ((0, 0), (0, SP_PAD - SP_STRIPS), (0, 0))).reshape(NW * SP_PAD, 128)
    j = np.arange(B * L, dtype=np.int32)
    b = j // L
    qdst = (((b // BPW) // NC) * BPW + (b % BPW)).astype(np.int32).reshape(NW, SQ_STRIPS, 128)
    qdst = np.pad(qdst, ((0, 0), (0, SQ_PAD - SQ_STRIPS), (0, 0))).reshape(NW * SQ_PAD, 128)
    return sdst, qdst


def _body(idx1, hist1, tsp, tsq, sdst2, qdst2, out,
          sidx, hidx, sdstv, qdstv, pdstv, rows, acc, shacc, sem):
    c = lax.axis_index("c")
    s = lax.axis_index("s")
    w = s * NC + c

    pltpu.sync_copy(idx1.at[pl.ds(pl.multiple_of(w * (SP_STRIPS * 128), 128), SP_STRIPS * 128)], sidx)
    pltpu.sync_copy(hist1.at[pl.ds(pl.multiple_of(w * (SQ_STRIPS * 128), 128), SQ_STRIPS * 128)], hidx)
    pltpu.sync_copy(sdst2.at[pl.ds(pl.multiple_of(w * SP_PAD, 8), SP_PAD)], sdstv)
    pltpu.sync_copy(qdst2.at[pl.ds(pl.multiple_of(w * SQ_PAD, 8), SQ_PAD)], qdstv)

    lane = jnp.arange(16, dtype=jnp.int32) * (F + 1)
    pbase = (w * BPW) * (F + 1) + F
    for q in range(BPW // 16):
        pdstv[pl.ds(q * 16, 16)] = pbase + q * 16 * (F + 1) + lane

    def _zero(r, carry):
        for q in range(NQ):
            acc[r, pl.ds(q * 16, 16)] = jnp.zeros((16,), jnp.float32)
        return carry
    lax.fori_loop(0, BPW, _zero, 0)
    pltpu.sync_copy(acc, shacc.at[pl.ds(s * BPW, BPW)])

    def _seq(t, carry):
        gi = hidx.at[pl.ds(pl.multiple_of(t * 128, 128), 128)]
        pltpu.async_copy(tsq.at[gi], rows, sem).wait()
        pltpu.sync_copy(rows, shacc.at[qdstv.at[t]], add=True)
        return carry
    lax.fori_loop(0, SQ_STRIPS, _seq, 0)

    pltpu.sync_copy(shacc.at[pl.ds(s * BPW, BPW)], acc)
    def _scale(r, carry):
        for q in range(NQ):
            acc[r, pl.ds(q * 16, 16)] = acc[r, pl.ds(q * 16, 16)] * (1.0 / L)
        return carry
    lax.fori_loop(0, BPW, _scale, 0)
    pltpu.async_copy(acc, out.at[pdstv], sem).wait()

    def _sp(t, carry):
        gi = sidx.at[pl.ds(pl.multiple_of(t * 128, 128), 128)]
        pltpu.async_copy(tsp.at[gi], rows, sem).wait()
        pltpu.async_copy(rows, out.at[sdstv.at[t]], sem).wait()
        return carry
    lax.fori_loop(0, SP_STRIPS, _sp, 0)


_sc_call = functools.partial(
    pl.kernel,
    out_type=jax.ShapeDtypeStruct((B * (F + 1), D), jnp.float32),
    mesh=plsc.VectorSubcoreMesh(core_axis_name="c", subcore_axis_name="s"),
    compiler_params=pltpu.CompilerParams(use_tc_tiling_on_sc=False),
    scratch_types=[
        pltpu.VMEM((SP_STRIPS * 128,), jnp.int32),
        pltpu.VMEM((SQ_STRIPS * 128,), jnp.int32),
        pltpu.VMEM((SP_PAD, 128), jnp.int32),
        pltpu.VMEM((SQ_PAD, 128), jnp.int32),
        pltpu.VMEM((BPW,), jnp.int32),
        pltpu.VMEM((128, D), jnp.float32),
        pltpu.VMEM((BPW, D), jnp.float32),
        pltpu.VMEM_SHARED((NS * BPW, D), jnp.float32),
        pltpu.SemaphoreType.DMA,
    ],
)(_body)


def kernel(indices, hist, table_sparse, table_seq):
    sdst, qdst = _dest_arrays()
    idx1 = indices.astype(jnp.int32).reshape(-1)
    hist1 = hist.astype(jnp.int32).reshape(-1)
    out = _sc_call(idx1, hist1, table_sparse, table_seq,
                   jnp.asarray(sdst), jnp.asarray(qdst))
    return out.reshape(B, (F + 1) * D)


# speedup vs baseline: 4.9197x; 1.1822x over previous
import functools

import numpy as np
import jax
import jax.numpy as jnp
from jax import lax
from jax.experimental import pallas as pl
from jax.experimental.pallas import tpu as pltpu
from jax.experimental.pallas import tpu_sc as plsc

B, F, L, V, D = 4096, 26, 50, 100000, 64
NC, NS = 2, 16
NW = NC * NS
BPW = B // NW
SP_STRIPS = BPW * F // 128
SQ_STRIPS = BPW * L // 128
SP_PAD = 32
SQ_PAD = 56
NQ = D // 16


@functools.lru_cache(maxsize=1)
def _dest_arrays():
    i = np.arange(B * F, dtype=np.int32)
    sdst = ((i // F) * (F + 1) + (i % F)).astype(np.int32).reshape(NW, SP_STRIPS, 128)
    sdst = np.pad(sdst, ((0, 0), (0, SP_PAD - SP_STRIPS), (0, 0))).reshape(NW * SP_PAD, 128)
    j = np.arange(B * L, dtype=np.int32)
    b = j // L
    qdst = (((b // BPW) // NC) * BPW + (b % BPW)).astype(np.int32).reshape(NW, SQ_STRIPS, 128)
    qdst = np.pad(qdst, ((0, 0), (0, SQ_PAD - SQ_STRIPS), (0, 0))).reshape(NW * SQ_PAD, 128)
    return sdst, qdst


def _body(idx1, hist1, tsp, tsq, sdst2, qdst2, out,
          sidx, hidx, sdstv, qdstv, pdstv, rows0, rows1, acc,
          shacc, g0, g1, s0, s1, p0):
    c = lax.axis_index("c")
    s = lax.axis_index("s")
    w = s * NC + c

    def drain(sem, dst):
        pltpu.make_async_copy(tsp.at[pl.ds(0, dst.shape[0])], dst, sem).wait()

    st0 = pltpu.async_copy(
        idx1.at[pl.ds(pl.multiple_of(w * (SP_STRIPS * 128), 128), SP_STRIPS * 128)], sidx, g0)
    st1 = pltpu.async_copy(
        hist1.at[pl.ds(pl.multiple_of(w * (SQ_STRIPS * 128), 128), SQ_STRIPS * 128)], hidx, g1)
    st2 = pltpu.async_copy(sdst2.at[pl.ds(pl.multiple_of(w * SP_PAD, 8), SP_PAD)], sdstv, s0)
    st3 = pltpu.async_copy(qdst2.at[pl.ds(pl.multiple_of(w * SQ_PAD, 8), SQ_PAD)], qdstv, s1)

    lane = jnp.arange(16, dtype=jnp.int32) * (F + 1)
    pbase = (w * BPW) * (F + 1) + F
    for q in range(BPW // 16):
        pdstv[pl.ds(q * 16, 16)] = pbase + q * 16 * (F + 1) + lane

    def _zero(r, carry):
        for q in range(NQ):
            acc[r, pl.ds(q * 16, 16)] = jnp.zeros((16,), jnp.float32)
        return carry
    lax.fori_loop(0, BPW, _zero, 0)
    st0.wait(); st1.wait(); st2.wait(); st3.wait()
    pltpu.sync_copy(acc, shacc.at[pl.ds(s * BPW, BPW)])

    def gath(tbl, t, dst, sem):
        gi = hidx.at[pl.ds(pl.multiple_of(t * 128, 128), 128)] if tbl is tsq else \
             sidx.at[pl.ds(pl.multiple_of(t * 128, 128), 128)]
        pltpu.async_copy(tbl.at[gi], dst, sem)

    gath(tsq, 0, rows0, g0)

    def _seq_pair(p, carry):
        t0 = 2 * p

        @pl.when(p > 0)
        def _():
            drain(s1, rows1)
        gath(tsq, t0 + 1, rows1, g1)
        drain(g0, rows0)
        pltpu.async_copy(rows0, shacc.at[qdstv.at[t0]], s0, add=True)

        @pl.when(p < SQ_STRIPS // 2 - 1)
        def _():
            drain(s0, rows0)
            gath(tsq, t0 + 2, rows0, g0)
        drain(g1, rows1)
        pltpu.async_copy(rows1, shacc.at[qdstv.at[t0 + 1]], s1, add=True)
        return carry
    lax.fori_loop(0, SQ_STRIPS // 2, _seq_pair, 0)
    drain(s0, rows0)
    drain(s1, rows1)

    pltpu.sync_copy(shacc.at[pl.ds(s * BPW, BPW)], acc)
    gath(tsp, 0, rows0, g0)

    def _scale(r, carry):
        for q in range(NQ):
            acc[r, pl.ds(q * 16, 16)] = acc[r, pl.ds(q * 16, 16)] * (1.0 / L)
        return carry
    lax.fori_loop(0, BPW, _scale, 0)
    pooled = pltpu.async_copy(acc, out.at[pdstv], p0)

    def _sp_pair(p, carry):
        t0 = 2 * p

        @pl.when(p > 0)
        def _():
            drain(s1, rows1)
        gath(tsp, t0 + 1, rows1, g1)
        drain(g0, rows0)
        pltpu.async_copy(rows0, out.at[sdstv.at[t0]], s0)

        @pl.when(p < SP_STRIPS // 2 - 1)
        def _():
            drain(s0, rows0)
            gath(tsp, t0 + 2, rows0, g0)
        drain(g1, rows1)
        pltpu.async_copy(rows1, out.at[sdstv.at[t0 + 1]], s1)
        return carry
    lax.fori_loop(0, SP_STRIPS // 2, _sp_pair, 0)
    drain(s0, rows0)
    drain(s1, rows1)
    pooled.wait()


_sc_call = functools.partial(
    pl.kernel,
    out_type=jax.ShapeDtypeStruct((B * (F + 1), D), jnp.float32),
    mesh=plsc.VectorSubcoreMesh(core_axis_name="c", subcore_axis_name="s"),
    compiler_params=pltpu.CompilerParams(use_tc_tiling_on_sc=False),
    scratch_types=[
        pltpu.VMEM((SP_STRIPS * 128,), jnp.int32),
        pltpu.VMEM((SQ_STRIPS * 128,), jnp.int32),
        pltpu.VMEM((SP_PAD, 128), jnp.int32),
        pltpu.VMEM((SQ_PAD, 128), jnp.int32),
        pltpu.VMEM((BPW,), jnp.int32),
        pltpu.VMEM((128, D), jnp.float32),
        pltpu.VMEM((128, D), jnp.float32),
        pltpu.VMEM((BPW, D), jnp.float32),
        pltpu.VMEM_SHARED((NS * BPW, D), jnp.float32),
        pltpu.SemaphoreType.DMA,
        pltpu.SemaphoreType.DMA,
        pltpu.SemaphoreType.DMA,
        pltpu.SemaphoreType.DMA,
        pltpu.SemaphoreType.DMA,
    ],
)(_body)


def kernel(indices, hist, table_sparse, table_seq):
    sdst, qdst = _dest_arrays()
    idx1 = indices.astype(jnp.int32).reshape(-1)
    hist1 = hist.astype(jnp.int32).reshape(-1)
    out = _sc_call(idx1, hist1, table_sparse, table_seq,
                   jnp.asarray(sdst), jnp.asarray(qdst))
    return out.reshape(B, (F + 1) * D)


# speedup vs baseline: 5.1623x; 1.0493x over previous
import functools

import numpy as np
import jax
import jax.numpy as jnp
from jax import lax
from jax.experimental import pallas as pl
from jax.experimental.pallas import tpu as pltpu
from jax.experimental.pallas import tpu_sc as plsc

B, F, L, V, D = 4096, 26, 50, 100000, 64
NC, NS = 2, 16
NW = NC * NS
BPW = B // NW
SP_STRIPS = BPW * F // 128
SQ_STRIPS = BPW * L // 128
SQ_PAD = 56
NQ = D // 16


@functools.lru_cache(maxsize=1)
def _qdst_array():
    j = np.arange(B * L, dtype=np.int32)
    b = j // L
    qdst = (((b // BPW) // NC) * BPW + (b % BPW)).astype(np.int32).reshape(NW, SQ_STRIPS, 128)
    qdst = np.pad(qdst, ((0, 0), (0, SQ_PAD - SQ_STRIPS), (0, 0))).reshape(NW * SQ_PAD, 128)
    return qdst


_MESH = plsc.VectorSubcoreMesh(core_axis_name="c", subcore_axis_name="s")
_PARAMS = pltpu.CompilerParams(use_tc_tiling_on_sc=False)


def _sparse_body(idx1, tsp, out, sidx, rows0, rows1, g0, g1, s0, s1):
    c = lax.axis_index("c")
    s = lax.axis_index("s")
    w = s * NC + c
    obase = w * (SP_STRIPS * 128)

    def drain(sem, dst):
        pltpu.make_async_copy(tsp.at[pl.ds(0, 128)], dst, sem).wait()

    pltpu.sync_copy(
        idx1.at[pl.ds(pl.multiple_of(obase, 128), SP_STRIPS * 128)], sidx)

    def gath(t, dst, sem):
        gi = sidx.at[pl.ds(pl.multiple_of(t * 128, 128), 128)]
        pltpu.async_copy(tsp.at[gi], dst, sem)

    def store(t, src, sem):
        pltpu.async_copy(
            src, out.at[pl.ds(pl.multiple_of(obase + t * 128, 128), 128)], sem)

    gath(0, rows0, g0)

    def _pair(p, carry):
        t0 = 2 * p

        @pl.when(p > 0)
        def _():
            drain(s1, rows1)
        gath(t0 + 1, rows1, g1)
        drain(g0, rows0)
        store(t0, rows0, s0)

        @pl.when(p < SP_STRIPS // 2 - 1)
        def _():
            drain(s0, rows0)
            gath(t0 + 2, rows0, g0)
        drain(g1, rows1)
        store(t0 + 1, rows1, s1)
        return carry
    lax.fori_loop(0, SP_STRIPS // 2, _pair, 0)
    drain(s0, rows0)
    drain(s1, rows1)


def _seq_body(hist1, tsq, qdst2, out, hidx, qdstv, rows0, rows1, acc,
              shacc, g0, g1, s0, s1):
    c = lax.axis_index("c")
    s = lax.axis_index("s")
    w = s * NC + c

    def drain(sem, dst):
        pltpu.make_async_copy(tsq.at[pl.ds(0, 128)], dst, sem).wait()

    st0 = pltpu.async_copy(
        hist1.at[pl.ds(pl.multiple_of(w * (SQ_STRIPS * 128), 128), SQ_STRIPS * 128)],
        hidx, g0)
    st1 = pltpu.async_copy(qdst2.at[pl.ds(pl.multiple_of(w * SQ_PAD, 8), SQ_PAD)], qdstv, g1)

    def _zero(r, carry):
        for q in range(NQ):
            acc[r, pl.ds(q * 16, 16)] = jnp.zeros((16,), jnp.float32)
        return carry
    lax.fori_loop(0, BPW, _zero, 0)
    st0.wait()
    st1.wait()
    pltpu.sync_copy(acc, shacc.at[pl.ds(s * BPW, BPW)])

    def gath(t, dst, sem):
        gi = hidx.at[pl.ds(pl.multiple_of(t * 128, 128), 128)]
        pltpu.async_copy(tsq.at[gi], dst, sem)

    gath(0, rows0, g0)

    def _pair(p, carry):
        t0 = 2 * p

        @pl.when(p > 0)
        def _():
            drain(s1, rows1)
        gath(t0 + 1, rows1, g1)
        drain(g0, rows0)
        pltpu.async_copy(rows0, shacc.at[qdstv.at[t0]], s0, add=True)

        @pl.when(p < SQ_STRIPS // 2 - 1)
        def _():
            drain(s0, rows0)
            gath(t0 + 2, rows0, g0)
        drain(g1, rows1)
        pltpu.async_copy(rows1, shacc.at[qdstv.at[t0 + 1]], s1, add=True)
        return carry
    lax.fori_loop(0, SQ_STRIPS // 2, _pair, 0)
    drain(s0, rows0)
    drain(s1, rows1)

    pltpu.sync_copy(shacc.at[pl.ds(s * BPW, BPW)], acc)

    def _scale(r, carry):
        for q in range(NQ):
            acc[r, pl.ds(q * 16, 16)] = acc[r, pl.ds(q * 16, 16)] * (1.0 / L)
        return carry
    lax.fori_loop(0, BPW, _scale, 0)
    pltpu.sync_copy(acc, out.at[pl.ds(pl.multiple_of(w * BPW, 128), BPW)])


_sparse_call = functools.partial(
    pl.kernel,
    out_type=jax.ShapeDtypeStruct((B * F, D), jnp.float32),
    mesh=_MESH,
    compiler_params=_PARAMS,
    scratch_types=[
        pltpu.VMEM((SP_STRIPS * 128,), jnp.int32),
        pltpu.VMEM((128, D), jnp.float32),
        pltpu.VMEM((128, D), jnp.float32),
        pltpu.SemaphoreType.DMA,
        pltpu.SemaphoreType.DMA,
        pltpu.SemaphoreType.DMA,
        pltpu.SemaphoreType.DMA,
    ],
)(_sparse_body)

_seq_call = functools.partial(
    pl.kernel,
    out_type=jax.ShapeDtypeStruct((B, D), jnp.float32),
    mesh=_MESH,
    compiler_params=_PARAMS,
    scratch_types=[
        pltpu.VMEM((SQ_STRIPS * 128,), jnp.int32),
        pltpu.VMEM((SQ_PAD, 128), jnp.int32),
        pltpu.VMEM((128, D), jnp.float32),
        pltpu.VMEM((128, D), jnp.float32),
        pltpu.VMEM((BPW, D), jnp.float32),
        pltpu.VMEM_SHARED((NS * BPW, D), jnp.float32),
        pltpu.SemaphoreType.DMA,
        pltpu.SemaphoreType.DMA,
        pltpu.SemaphoreType.DMA,
        pltpu.SemaphoreType.DMA,
    ],
)(_seq_body)


def kernel(indices, hist, table_sparse, table_seq):
    idx1 = indices.astype(jnp.int32).reshape(-1)
    hist1 = hist.astype(jnp.int32).reshape(-1)
    out_sp = _sparse_call(idx1, table_sparse)
    out_pool = _seq_call(hist1, table_seq, jnp.asarray(_qdst_array()))
    return jnp.concatenate([out_sp.reshape(B, F * D), out_pool], axis=-1)


# speedup vs baseline: 5.2388x; 1.0148x over previous
import functools

import numpy as np
import jax
import jax.numpy as jnp
from jax import lax
from jax.experimental import pallas as pl
from jax.experimental.pallas import tpu as pltpu
from jax.experimental.pallas import tpu_sc as plsc

B, F, L, V, D = 4096, 26, 50, 100000, 64
NC, NS = 2, 16
NW = NC * NS
BPW = B // NW
BH = B // 2
BPWH = BH // NW
SP_STRIPS = BPWH * F // 128
SQ_STRIPS = BPW * L // 128
SQ_PAD = 56
NQ = D // 16


@functools.lru_cache(maxsize=1)
def _qdst_array():
    j = np.arange(B * L, dtype=np.int32)
    b = j // L
    qdst = (((b // BPW) // NC) * BPW + (b % BPW)).astype(np.int32).reshape(NW, SQ_STRIPS, 128)
    qdst = np.pad(qdst, ((0, 0), (0, SQ_PAD - SQ_STRIPS), (0, 0))).reshape(NW * SQ_PAD, 128)
    return qdst


_MESH = plsc.VectorSubcoreMesh(core_axis_name="c", subcore_axis_name="s")
_PARAMS = pltpu.CompilerParams(use_tc_tiling_on_sc=False)


def _make_sparse_body(sample_base):
    def _sparse_body(idx1, tsp, out, sidx, rows0, rows1, g0, g1, s0, s1):
        c = lax.axis_index("c")
        s = lax.axis_index("s")
        w = s * NC + c
        ibase = sample_base * F + w * (SP_STRIPS * 128)
        obase = w * (SP_STRIPS * 128)

        def drain(sem, dst):
            pltpu.make_async_copy(tsp.at[pl.ds(0, 128)], dst, sem).wait()

        pltpu.sync_copy(
            idx1.at[pl.ds(pl.multiple_of(ibase, 128), SP_STRIPS * 128)], sidx)

        def gath(t, dst, sem):
            gi = sidx.at[pl.ds(pl.multiple_of(t * 128, 128), 128)]
            pltpu.async_copy(tsp.at[gi], dst, sem)

        def store(t, src, sem):
            pltpu.async_copy(
                src, out.at[pl.ds(pl.multiple_of(obase + t * 128, 128), 128)], sem)

        gath(0, rows0, g0)

        def _pair(p, carry):
            t0 = 2 * p

            @pl.when(p > 0)
            def _():
                drain(s1, rows1)
            gath(t0 + 1, rows1, g1)
            drain(g0, rows0)
            store(t0, rows0, s0)

            @pl.when(p < SP_STRIPS // 2 - 1)
            def _():
                drain(s0, rows0)
                gath(t0 + 2, rows0, g0)
            drain(g1, rows1)
            store(t0 + 1, rows1, s1)
            return carry
        lax.fori_loop(0, SP_STRIPS // 2, _pair, 0)
        drain(s1, rows1)
        gath(SP_STRIPS - 1, rows1, g1)
        drain(s0, rows0)
        drain(g1, rows1)
        store(SP_STRIPS - 1, rows1, s1)
        drain(s1, rows1)
    return _sparse_body


def _seq_body(hist1, tsq, qdst2, out, hidx, qdstv, rows0, rows1, acc,
              shacc, g0, g1, s0, s1):
    c = lax.axis_index("c")
    s = lax.axis_index("s")
    w = s * NC + c

    def drain(sem, dst):
        pltpu.make_async_copy(tsq.at[pl.ds(0, 128)], dst, sem).wait()

    st0 = pltpu.async_copy(
        hist1.at[pl.ds(pl.multiple_of(w * (SQ_STRIPS * 128), 128), SQ_STRIPS * 128)],
        hidx, g0)
    st1 = pltpu.async_copy(qdst2.at[pl.ds(pl.multiple_of(w * SQ_PAD, 8), SQ_PAD)], qdstv, g1)

    def _zero(r, carry):
        for q in range(NQ):
            acc[r, pl.ds(q * 16, 16)] = jnp.zeros((16,), jnp.float32)
        return carry
    lax.fori_loop(0, BPW, _zero, 0)
    st0.wait()
    st1.wait()
    pltpu.sync_copy(acc, shacc.at[pl.ds(s * BPW, BPW)])

    def gath(t, dst, sem):
        gi = hidx.at[pl.ds(pl.multiple_of(t * 128, 128), 128)]
        pltpu.async_copy(tsq.at[gi], dst, sem)

    gath(0, rows0, g0)

    def _pair(p, carry):
        t0 = 2 * p

        @pl.when(p > 0)
        def _():
            drain(s1, rows1)
        gath(t0 + 1, rows1, g1)
        drain(g0, rows0)
        pltpu.async_copy(rows0, shacc.at[qdstv.at[t0]], s0, add=True)

        @pl.when(p < SQ_STRIPS // 2 - 1)
        def _():
            drain(s0, rows0)
            gath(t0 + 2, rows0, g0)
        drain(g1, rows1)
        pltpu.async_copy(rows1, shacc.at[qdstv.at[t0 + 1]], s1, add=True)
        return carry
    lax.fori_loop(0, SQ_STRIPS // 2, _pair, 0)
    drain(s0, rows0)
    drain(s1, rows1)

    pltpu.sync_copy(shacc.at[pl.ds(s * BPW, BPW)], acc)

    def _scale(r, carry):
        for q in range(NQ):
            acc[r, pl.ds(q * 16, 16)] = acc[r, pl.ds(q * 16, 16)] * (1.0 / L)
        return carry
    lax.fori_loop(0, BPW, _scale, 0)
    pltpu.sync_copy(acc, out.at[pl.ds(pl.multiple_of(w * BPW, 128), BPW)])


def _make_sparse_call(sample_base):
    return functools.partial(
        pl.kernel,
        out_type=jax.ShapeDtypeStruct((BH * F, D), jnp.float32),
        mesh=_MESH,
        compiler_params=_PARAMS,
        scratch_types=[
            pltpu.VMEM((SP_STRIPS * 128,), jnp.int32),
            pltpu.VMEM((128, D), jnp.float32),
            pltpu.VMEM((128, D), jnp.float32),
            pltpu.SemaphoreType.DMA,
            pltpu.SemaphoreType.DMA,
            pltpu.SemaphoreType.DMA,
            pltpu.SemaphoreType.DMA,
        ],
    )(_make_sparse_body(sample_base))


_sparse_call_0 = _make_sparse_call(0)
_sparse_call_1 = _make_sparse_call(BH)

_seq_call = functools.partial(
    pl.kernel,
    out_type=jax.ShapeDtypeStruct((B, D), jnp.float32),
    mesh=_MESH,
    compiler_params=_PARAMS,
    scratch_types=[
        pltpu.VMEM((SQ_STRIPS * 128,), jnp.int32),
        pltpu.VMEM((SQ_PAD, 128), jnp.int32),
        pltpu.VMEM((128, D), jnp.float32),
        pltpu.VMEM((128, D), jnp.float32),
        pltpu.VMEM((BPW, D), jnp.float32),
        pltpu.VMEM_SHARED((NS * BPW, D), jnp.float32),
        pltpu.SemaphoreType.DMA,
        pltpu.SemaphoreType.DMA,
        pltpu.SemaphoreType.DMA,
        pltpu.SemaphoreType.DMA,
    ],
)(_seq_body)


def kernel(indices, hist, table_sparse, table_seq):
    idx1 = indices.astype(jnp.int32).reshape(-1)
    hist1 = hist.astype(jnp.int32).reshape(-1)
    sp1 = _sparse_call_0(idx1, table_sparse)
    sp2 = _sparse_call_1(idx1, table_sparse)
    pool = _seq_call(hist1, table_seq, jnp.asarray(_qdst_array()))
    sp = jnp.concatenate(
        [sp1.reshape(BH, F * D), sp2.reshape(BH, F * D)], axis=0)
    return jnp.concatenate([sp, pool], axis=-1)
